# baseline (speedup 1.0000x reference)
import functools
import numpy as np
import jax
import jax.numpy as jnp
from jax import lax
from jax.experimental import pallas as pl
from jax.experimental.pallas import tpu as pltpu
from jax.experimental.pallas import tpu_sc as plsc

_N, _C, _H, _W = 4, 96, 64, 64
_CP = 128
_HW = _H * _W
_RP = 1024
_KTC = 768
_RSC = _RP - _KTC
_RB = 128
_NC, _NS, _L = 2, 16, 16
_RPT = _RSC // (_NC * _NS)
_KC = _C // _L
_NEG = np.float32(-1e30)


def _sc_body(inp_hbm, b_hbm, x1_hbm, y1_hbm, x2_hbm, y2_hbm,
             out_hbm, bv, x1v, y1v, x2v, y2v,
             idx0a, idx1a, idx0b, idx1b, patcha, patchb,
             sbuf, wbuf, outbuf, sema, semb):
    wid = lax.axis_index("s") * _NC + lax.axis_index("c")
    base = wid * _RPT

    pltpu.sync_copy(b_hbm.at[pl.ds(base, _RPT)], bv)
    pltpu.sync_copy(x1_hbm.at[pl.ds(base, _RPT)], x1v)
    pltpu.sync_copy(y1_hbm.at[pl.ds(base, _RPT)], y1v)
    pltpu.sync_copy(x2_hbm.at[pl.ds(base, _RPT)], x2v)
    pltpu.sync_copy(y2_hbm.at[pl.ds(base, _RPT)], y2v)

    lane = lax.broadcasted_iota(jnp.int32, (_L,), 0)
    c96 = jnp.full((_L,), _C, jnp.int32)

    def issue(r, idx0, idx1, patch, sem):
        rv = jnp.full((_L,), r, jnp.int32)
        b_b = plsc.load_gather(bv, [rv])
        x1_b = plsc.load_gather(x1v, [rv])
        y1_b = plsc.load_gather(y1v, [rv])
        h_b = plsc.load_gather(y2v, [rv]) - y1_b
        h_s = jnp.max(h_b)
        basev = (b_b * _H + y1_b) * _W + x1_b
        for j in range(16):
            idx_j = jnp.clip(basev + j * _W + lane, 0, _N * _HW - 1)
            if j < 8:
                idx0[pl.ds(j * _L, _L)] = idx_j
            else:
                idx1[pl.ds((j - 8) * _L, _L)] = idx_j
        cp0 = pltpu.async_copy(inp_hbm.at[idx0], patch.at[pl.ds(0, 128)], sem)

        @pl.when(h_s > 8)
        def _():
            pltpu.async_copy(inp_hbm.at[idx1], patch.at[pl.ds(128, 128)], sem)
        return cp0, h_s

    def wait2(idx1, patch, sem, cp0, h_s):
        cp0.wait()

        @pl.when(h_s > 8)
        def _():
            pltpu.make_async_copy(
                inp_hbm.at[idx1], patch.at[pl.ds(128, 128)], sem).wait()

    def compute(r, patch, h_s):
        rv = jnp.full((_L,), r, jnp.int32)
        x1_b = plsc.load_gather(x1v, [rv])
        y1_b = plsc.load_gather(y1v, [rv])
        w_b = plsc.load_gather(x2v, [rv]) - x1_b
        h_b = plsc.load_gather(y2v, [rv]) - y1_b

        lmask = lane < w_b

        def mbody(j, mvec):
            s_j = plsc.load_gather(patch, [lane + j * _L, c96])
            sm_j = jnp.where(lmask, s_j, _NEG)
            sbuf[pl.ds(pl.multiple_of(j * _L, _L), _L)] = sm_j
            return jnp.maximum(mvec, sm_j)

        mvec = lax.fori_loop(0, h_s, mbody, jnp.full((_L,), _NEG))
        mb = jnp.full((_L,), jnp.max(mvec))

        def ebody(j, dvec):
            off = pl.multiple_of(j * _L, _L)
            e_j = jnp.exp(sbuf[pl.ds(off, _L)] - mb)
            wbuf[pl.ds(off, _L)] = e_j
            return dvec + e_j

        dvec = lax.fori_loop(0, h_s, ebody, jnp.zeros((_L,), jnp.float32))

        def jbody(j, accs):
            accs = list(accs)
            for l in range(16):
                p = j * 16 + l
                wb = plsc.load_gather(wbuf, [jnp.full((_L,), p, jnp.int32)])
                for k in range(_KC):
                    accs[k] = accs[k] + wb * patch[p, pl.ds(k * _L, _L)]
            return tuple(accs)

        accs = lax.fori_loop(
            0, h_s, jbody,
            tuple(jnp.zeros((_L,), jnp.float32) for _ in range(_KC)))

        db = jnp.full((_L,), jnp.sum(dvec))
        vvec = (w_b > 0) & (h_b > 0) & (db > 0.0)
        invb = jnp.where(vvec, 1.0 / jnp.where(vvec, db, 1.0),
                         jnp.float32(0.0))
        for k in range(_KC):
            outbuf[r, pl.ds(k * _L, _L)] = accs[k] * invb

    def pair_body(i, _):
        r0 = 2 * i
        cpa0, ha = issue(r0, idx0a, idx1a, patcha, sema)
        cpb0, hb = issue(r0 + 1, idx0b, idx1b, patchb, semb)
        wait2(idx1a, patcha, sema, cpa0, ha)
        compute(r0, patcha, ha)
        wait2(idx1b, patchb, semb, cpb0, hb)
        compute(r0 + 1, patchb, hb)
        return ()

    lax.fori_loop(0, _RPT // 2, pair_body, ())
    pltpu.sync_copy(outbuf, out_hbm.at[pl.ds(base, _RPT)])


def _tc_body(rois_ref, score_ref, inp_ref, out_ref):
    rois = rois_ref[...]
    bi = rois[:, 0:1].astype(jnp.int32)
    x1 = jnp.round(rois[:, 1:2]).astype(jnp.int32)
    y1 = jnp.round(rois[:, 2:3]).astype(jnp.int32)
    x2 = jnp.round(rois[:, 3:4]).astype(jnp.int32)
    y2 = jnp.round(rois[:, 4:5]).astype(jnp.int32)

    hw = jax.lax.broadcasted_iota(jnp.int32, (_RB, _HW), 1)
    px = hw & (_W - 1)
    py = hw >> 6
    inside = (py >= y1) & (py < y2) & (px >= x1) & (px < x2)

    nid = jax.lax.broadcasted_iota(jnp.int32, (_RB, _N), 1)
    onehot = (bi == nid).astype(jnp.float32)
    s = jax.lax.dot_general(
        onehot, score_ref[...], (((1,), (0,)), ((), ())),
        preferred_element_type=jnp.float32,
        precision=jax.lax.Precision.HIGHEST)

    ms = jnp.where(inside, s, _NEG)
    m = jnp.max(ms, axis=1, keepdims=True)
    e = jnp.where(inside, jnp.exp(ms - m), jnp.float32(0.0))
    denom = jnp.sum(e, axis=1, keepdims=True)

    acc = jnp.zeros((_RB, _C), jnp.float32)
    for n in range(_N):
        en = jnp.where(bi == n, e, jnp.float32(0.0))
        acc = acc + jax.lax.dot_general(
            en, inp_ref[n][:, :_C], (((1,), (0,)), ((), ())),
            preferred_element_type=jnp.float32,
            precision=jax.lax.Precision.HIGHEST)

    valid = (x1 < x2) & (y1 < y2)
    scale = jnp.where((denom > 0.0) & valid, 1.0 / denom, jnp.float32(0.0))
    out_ref[...] = acc * scale


@jax.jit
def _call(inp_rows, score2, rois_tc, b, x1, y1, x2, y2):
    mesh = plsc.VectorSubcoreMesh(
        core_axis_name="c", subcore_axis_name="s",
        num_cores=_NC, num_subcores=_NS)
    sc = functools.partial(
        pl.kernel, mesh=mesh,
        compiler_params=pltpu.CompilerParams(needs_layout_passes=False),
        out_type=jax.ShapeDtypeStruct((_RSC, _C), jnp.float32),
        scratch_types=[
            pltpu.VMEM((_RPT,), jnp.int32),
            pltpu.VMEM((_RPT,), jnp.int32),
            pltpu.VMEM((_RPT,), jnp.int32),
            pltpu.VMEM((_RPT,), jnp.int32),
            pltpu.VMEM((_RPT,), jnp.int32),
            pltpu.VMEM((128,), jnp.int32),
            pltpu.VMEM((128,), jnp.int32),
            pltpu.VMEM((128,), jnp.int32),
            pltpu.VMEM((128,), jnp.int32),
            pltpu.VMEM((256, _CP), jnp.float32),
            pltpu.VMEM((256, _CP), jnp.float32),
            pltpu.VMEM((256,), jnp.float32),
            pltpu.VMEM((256,), jnp.float32),
            pltpu.VMEM((_RPT, _C), jnp.float32),
            pltpu.SemaphoreType.DMA,
            pltpu.SemaphoreType.DMA,
        ])(_sc_body)
    out_sc = sc(inp_rows, b, x1, y1, x2, y2)

    inp3 = inp_rows.reshape(_N, _HW, _CP)
    out_tc = pl.pallas_call(
        _tc_body,
        grid=(_KTC // _RB,),
        in_specs=[
            pl.BlockSpec((_RB, 5), lambda i: (i, 0)),
            pl.BlockSpec((_N, _HW), lambda i: (0, 0)),
            pl.BlockSpec((_N, _HW, _CP), lambda i: (0, 0, 0)),
        ],
        out_specs=pl.BlockSpec((_RB, _C), lambda i: (i, 0)),
        out_shape=jax.ShapeDtypeStruct((_KTC, _C), jnp.float32),
    )(rois_tc, score2, inp3)

    return jnp.concatenate([out_tc, out_sc], axis=0)


def kernel(input, rois, score_map):
    N, C, H, W = input.shape
    R = rois.shape[0]

    inp_t = jnp.transpose(input, (0, 2, 3, 1)).reshape(N * H * W, C)
    score_col = score_map.reshape(N * H * W, 1)
    pad = jnp.zeros((N * H * W, _CP - C - 1), jnp.float32)
    inp_rows = jnp.concatenate([inp_t, score_col, pad], axis=1)
    score2 = score_map.reshape(N, _HW)

    rois_p = jnp.zeros((_RP, 5), jnp.float32).at[:R].set(rois)
    rois_tc = rois_p[:_KTC]
    ri = jnp.round(rois_p[_KTC:]).astype(jnp.int32)
    b, x1, y1, x2, y2 = [ri[:, i] for i in range(5)]

    out = _call(inp_rows, score2, rois_tc, b, x1, y1, x2, y2)
    return out[:R].reshape(R, C, 1, 1)


# speedup vs baseline: 1.0438x; 1.0438x over previous
import functools
import numpy as np
import jax
import jax.numpy as jnp
from jax import lax
from jax.experimental import pallas as pl
from jax.experimental.pallas import tpu as pltpu
from jax.experimental.pallas import tpu_sc as plsc

_N, _C, _H, _W = 4, 96, 64, 64
_CP = 128
_HW = _H * _W
_RP = 1024
_NC, _NS, _L = 2, 16, 16
_RPT = _RP // (_NC * _NS)
_KC = _C // _L
_NEG = np.float32(-1e30)


def _sc_body(inp_hbm, b_hbm, x1_hbm, y1_hbm, x2_hbm, y2_hbm,
             out_hbm, bv, x1v, y1v, x2v, y2v,
             idx0a, idx1a, idx0b, idx1b, patcha, patchb,
             sbuf, wbuf, outbuf, sema, semb):
    wid = lax.axis_index("s") * _NC + lax.axis_index("c")
    base = wid * _RPT

    pltpu.sync_copy(b_hbm.at[pl.ds(base, _RPT)], bv)
    pltpu.sync_copy(x1_hbm.at[pl.ds(base, _RPT)], x1v)
    pltpu.sync_copy(y1_hbm.at[pl.ds(base, _RPT)], y1v)
    pltpu.sync_copy(x2_hbm.at[pl.ds(base, _RPT)], x2v)
    pltpu.sync_copy(y2_hbm.at[pl.ds(base, _RPT)], y2v)

    lane = lax.broadcasted_iota(jnp.int32, (_L,), 0)
    c96 = jnp.full((_L,), _C, jnp.int32)

    def issue(r, idx0, idx1, patch, sem):
        rv = jnp.full((_L,), r, jnp.int32)
        b_b = plsc.load_gather(bv, [rv])
        x1_b = plsc.load_gather(x1v, [rv])
        y1_b = plsc.load_gather(y1v, [rv])
        h_b = plsc.load_gather(y2v, [rv]) - y1_b
        h_s = jnp.max(h_b)
        basev = (b_b * _H + y1_b) * _W + x1_b
        for j in range(16):
            idx_j = jnp.clip(basev + j * _W + lane, 0, _N * _HW - 1)
            if j < 8:
                idx0[pl.ds(j * _L, _L)] = idx_j
            else:
                idx1[pl.ds((j - 8) * _L, _L)] = idx_j
        cp0 = pltpu.async_copy(inp_hbm.at[idx0], patch.at[pl.ds(0, 128)], sem)

        @pl.when(h_s > 8)
        def _():
            pltpu.async_copy(inp_hbm.at[idx1], patch.at[pl.ds(128, 128)], sem)
        return cp0, h_s

    def wait2(idx1, patch, sem, cp0, h_s):
        cp0.wait()

        @pl.when(h_s > 8)
        def _():
            pltpu.make_async_copy(
                inp_hbm.at[idx1], patch.at[pl.ds(128, 128)], sem).wait()

    def compute(r, patch, h_s):
        rv = jnp.full((_L,), r, jnp.int32)
        x1_b = plsc.load_gather(x1v, [rv])
        y1_b = plsc.load_gather(y1v, [rv])
        w_b = plsc.load_gather(x2v, [rv]) - x1_b
        h_b = plsc.load_gather(y2v, [rv]) - y1_b

        lmask = lane < w_b

        def mbody(j, mvec):
            s_j = plsc.load_gather(patch, [lane + j * _L, c96])
            sm_j = jnp.where(lmask, s_j, _NEG)
            sbuf[pl.ds(pl.multiple_of(j * _L, _L), _L)] = sm_j
            return jnp.maximum(mvec, sm_j)

        mvec = lax.fori_loop(0, h_s, mbody, jnp.full((_L,), _NEG))
        mb = jnp.full((_L,), jnp.max(mvec))

        def ebody(j, dvec):
            off = pl.multiple_of(j * _L, _L)
            e_j = jnp.exp(sbuf[pl.ds(off, _L)] - mb)
            wbuf[pl.ds(off, _L)] = e_j
            return dvec + e_j

        dvec = lax.fori_loop(0, h_s, ebody, jnp.zeros((_L,), jnp.float32))

        def jbody(j, accs):
            accs = list(accs)
            for l in range(16):
                p = j * 16 + l
                wb = plsc.load_gather(wbuf, [jnp.full((_L,), p, jnp.int32)])
                for k in range(_KC):
                    accs[k] = accs[k] + wb * patch[p, pl.ds(k * _L, _L)]
            return tuple(accs)

        accs = lax.fori_loop(
            0, h_s, jbody,
            tuple(jnp.zeros((_L,), jnp.float32) for _ in range(_KC)))

        db = jnp.full((_L,), jnp.sum(dvec))
        vvec = (w_b > 0) & (h_b > 0) & (db > 0.0)
        invb = jnp.where(vvec, 1.0 / jnp.where(vvec, db, 1.0),
                         jnp.float32(0.0))
        for k in range(_KC):
            outbuf[r, pl.ds(k * _L, _L)] = accs[k] * invb

    def pair_body(i, _):
        r0 = 2 * i
        cpa0, ha = issue(r0, idx0a, idx1a, patcha, sema)
        cpb0, hb = issue(r0 + 1, idx0b, idx1b, patchb, semb)
        wait2(idx1a, patcha, sema, cpa0, ha)
        compute(r0, patcha, ha)
        wait2(idx1b, patchb, semb, cpb0, hb)
        compute(r0 + 1, patchb, hb)
        return ()

    lax.fori_loop(0, _RPT // 2, pair_body, ())
    pltpu.sync_copy(outbuf, out_hbm.at[pl.ds(base, _RPT)])


@jax.jit
def _sc_call(inp_rows, b, x1, y1, x2, y2):
    mesh = plsc.VectorSubcoreMesh(
        core_axis_name="c", subcore_axis_name="s",
        num_cores=_NC, num_subcores=_NS)
    f = functools.partial(
        pl.kernel, mesh=mesh,
        compiler_params=pltpu.CompilerParams(needs_layout_passes=False),
        out_type=jax.ShapeDtypeStruct((_RP, _C), jnp.float32),
        scratch_types=[
            pltpu.VMEM((_RPT,), jnp.int32),
            pltpu.VMEM((_RPT,), jnp.int32),
            pltpu.VMEM((_RPT,), jnp.int32),
            pltpu.VMEM((_RPT,), jnp.int32),
            pltpu.VMEM((_RPT,), jnp.int32),
            pltpu.VMEM((128,), jnp.int32),
            pltpu.VMEM((128,), jnp.int32),
            pltpu.VMEM((128,), jnp.int32),
            pltpu.VMEM((128,), jnp.int32),
            pltpu.VMEM((256, _CP), jnp.float32),
            pltpu.VMEM((256, _CP), jnp.float32),
            pltpu.VMEM((256,), jnp.float32),
            pltpu.VMEM((256,), jnp.float32),
            pltpu.VMEM((_RPT, _C), jnp.float32),
            pltpu.SemaphoreType.DMA,
            pltpu.SemaphoreType.DMA,
        ])(_sc_body)
    return f(inp_rows, b, x1, y1, x2, y2)


def kernel(input, rois, score_map):
    N, C, H, W = input.shape
    R = rois.shape[0]

    inp_t = jnp.transpose(input, (0, 2, 3, 1)).reshape(N * H * W, C)
    score_col = score_map.reshape(N * H * W, 1)
    pad = jnp.zeros((N * H * W, _CP - C - 1), jnp.float32)
    inp_rows = jnp.concatenate([inp_t, score_col, pad], axis=1)

    ri = jnp.round(rois).astype(jnp.int32)
    ri = jnp.zeros((_RP, 5), jnp.int32).at[:R].set(ri)
    b, x1, y1, x2, y2 = [ri[:, i] for i in range(5)]

    out = _sc_call(inp_rows, b, x1, y1, x2, y2)
    return out[:R].reshape(R, C, 1, 1)
